# Initial kernel scaffold; baseline (speedup 1.0000x reference)
import functools

import jax
import jax.numpy as jnp
from jax import lax
from jax.experimental import pallas as pl
from jax.experimental.pallas import tpu as pltpu
from jax.experimental.pallas import tpu_sc as plsc

N_NODES = 10000
M_EDGES = 2500
E_PAIRS = 320000
F_DIM = 128

NC = 2
NS = 16
NW = NC * NS
CH = 80
ROWS_IDX = E_PAIRS // CH
CPW = ROWS_IDX // NW
MPAD = 2560
M_STRIPE = MPAD // NS
N_STRIPE = N_NODES // NS


def _matmul(x, w):
    def body(x_ref, w_ref, o_ref):
        o_ref[...] = jnp.dot(x_ref[...], w_ref[...],
                             preferred_element_type=jnp.float32)

    return pl.pallas_call(
        body,
        grid=(10,),
        in_specs=[
            pl.BlockSpec((N_NODES // 10, 128), lambda i: (i, 0)),
            pl.BlockSpec((128, F_DIM), lambda i: (0, 0)),
        ],
        out_specs=pl.BlockSpec((N_NODES // 10, F_DIM), lambda i: (i, 0)),
        out_shape=jax.ShapeDtypeStruct((N_NODES, F_DIM), jnp.float32),
    )(x, w)


_MESH = plsc.VectorSubcoreMesh(core_axis_name="c", subcore_axis_name="s",
                               num_cores=NC, num_subcores=NS)


@functools.partial(
    pl.kernel,
    out_type=jax.ShapeDtypeStruct((NC, MPAD, F_DIM), jnp.float32),
    mesh=_MESH,
    scratch_types=[
        pltpu.VMEM((CPW, CH), jnp.int32),
        pltpu.VMEM((CPW, CH), jnp.int32),
        pltpu.VMEM((CH, F_DIM), jnp.float32),
        pltpu.VMEM_SHARED((MPAD, F_DIM), jnp.float32),
    ],
)
def _stage1(xp_hbm, g1s_hbm, g1d_hbm, zeros_hbm, out_hbm,
            sidx, didx, rows, acc):
    c = lax.axis_index("c")
    s = lax.axis_index("s")
    w = c * NS + s

    pltpu.sync_copy(zeros_hbm.at[pl.ds(s * M_STRIPE, M_STRIPE)],
                    acc.at[pl.ds(s * M_STRIPE, M_STRIPE)])
    pltpu.sync_copy(g1s_hbm.at[pl.ds(w * CPW, CPW)], sidx)
    pltpu.sync_copy(g1d_hbm.at[pl.ds(w * CPW, CPW)], didx)
    plsc.subcore_barrier()

    def chunk(j, carry):
        pltpu.sync_copy(xp_hbm.at[sidx.at[j]], rows)
        pltpu.sync_copy(rows, acc.at[didx.at[j]], add=True)
        return carry

    lax.fori_loop(0, CPW, chunk, 0)
    plsc.subcore_barrier()
    pltpu.sync_copy(acc.at[pl.ds(s * M_STRIPE, M_STRIPE)],
                    out_hbm.at[c, pl.ds(s * M_STRIPE, M_STRIPE)])


@functools.partial(
    pl.kernel,
    out_type=jax.ShapeDtypeStruct((NC, N_NODES, F_DIM), jnp.float32),
    mesh=_MESH,
    scratch_types=[
        pltpu.VMEM((M_STRIPE, F_DIM), jnp.float32),
        pltpu.VMEM((M_STRIPE, F_DIM), jnp.float32),
        pltpu.VMEM((M_STRIPE,), jnp.float32),
        pltpu.VMEM((M_STRIPE,), jnp.float32),
        pltpu.VMEM((CPW, CH), jnp.int32),
        pltpu.VMEM((CPW, CH), jnp.int32),
        pltpu.VMEM((CH, F_DIM), jnp.float32),
        pltpu.VMEM_SHARED((MPAD, F_DIM), jnp.float32),
        pltpu.VMEM_SHARED((N_NODES, F_DIM), jnp.float32),
    ],
)
def _stage2(xep_hbm, dege_hbm, wmul_hbm, g2s_hbm, g2d_hbm, zeros_hbm, out_hbm,
            p0, p1, dege, wmul, sidx, didx, rows, xe, xv):
    c = lax.axis_index("c")
    s = lax.axis_index("s")
    w = c * NS + s

    mbase = s * M_STRIPE
    pltpu.sync_copy(xep_hbm.at[0, pl.ds(mbase, M_STRIPE)], p0)
    pltpu.sync_copy(xep_hbm.at[1, pl.ds(mbase, M_STRIPE)], p1)
    pltpu.sync_copy(dege_hbm.at[pl.ds(mbase, M_STRIPE)], dege)
    pltpu.sync_copy(wmul_hbm.at[pl.ds(mbase, M_STRIPE)], wmul)
    pltpu.sync_copy(zeros_hbm.at[pl.ds(s * N_STRIPE, N_STRIPE)],
                    xv.at[pl.ds(s * N_STRIPE, N_STRIPE)])

    def scale_row(r, carry):
        ridx = jnp.zeros((16,), jnp.int32) + r
        srow = (plsc.load_gather(dege, [ridx]) *
                plsc.load_gather(wmul, [ridx]))
        for cg in range(F_DIM // 16):
            sl = pl.ds(cg * 16, 16)
            p0[r, sl] = (p0[r, sl] + p1[r, sl]) * srow
        return carry

    lax.fori_loop(0, M_STRIPE, scale_row, 0)
    pltpu.sync_copy(p0, xe.at[pl.ds(mbase, M_STRIPE)])
    pltpu.sync_copy(g2s_hbm.at[pl.ds(w * CPW, CPW)], sidx)
    pltpu.sync_copy(g2d_hbm.at[pl.ds(w * CPW, CPW)], didx)
    plsc.subcore_barrier()

    def chunk(j, carry):
        pltpu.sync_copy(xe.at[sidx.at[j]], rows)
        pltpu.sync_copy(rows, xv.at[didx.at[j]], add=True)
        return carry

    lax.fori_loop(0, CPW, chunk, 0)
    plsc.subcore_barrier()
    pltpu.sync_copy(xv.at[pl.ds(s * N_STRIPE, N_STRIPE)],
                    out_hbm.at[c, pl.ds(s * N_STRIPE, N_STRIPE)])


def _combine(p0, p1, degv):
    def body(a_ref, b_ref, d_ref, o_ref):
        o_ref[...] = (a_ref[...] + b_ref[...]) * d_ref[...]

    blk = N_NODES // 10
    return pl.pallas_call(
        body,
        grid=(10,),
        in_specs=[
            pl.BlockSpec((blk, F_DIM), lambda i: (i, 0)),
            pl.BlockSpec((blk, F_DIM), lambda i: (i, 0)),
            pl.BlockSpec((blk, 1), lambda i: (i, 0)),
        ],
        out_specs=pl.BlockSpec((blk, F_DIM), lambda i: (i, 0)),
        out_shape=jax.ShapeDtypeStruct((N_NODES, F_DIM), jnp.float32),
    )(p0, p1, degv)


def kernel(X, g1_src, g1_dst, g2_src, g2_dst, W_lin, degE, degV, W):
    xp = _matmul(X, W_lin)

    g1s = g1_src.astype(jnp.int32).reshape(ROWS_IDX, CH)
    g1d = g1_dst.astype(jnp.int32).reshape(ROWS_IDX, CH)
    g2s = g2_src.astype(jnp.int32).reshape(ROWS_IDX, CH)
    g2d = g2_dst.astype(jnp.int32).reshape(ROWS_IDX, CH)

    zeros_m = jnp.zeros((MPAD, F_DIM), jnp.float32)
    xep = _stage1(xp, g1s, g1d, zeros_m)

    dege_p = jnp.pad(degE.reshape(M_EDGES), (0, MPAD - M_EDGES))
    wmul_p = jnp.pad(W.reshape(M_EDGES), (0, MPAD - M_EDGES))
    zeros_n = jnp.zeros((N_NODES, F_DIM), jnp.float32)
    xvp = _stage2(xep, dege_p, wmul_p, g2s, g2d, zeros_n)

    return _combine(xvp[0], xvp[1], degV)


# speedup vs baseline: 6.6712x; 6.6712x over previous
import functools

import jax
import jax.numpy as jnp
from jax import lax
from jax.experimental import pallas as pl
from jax.experimental.pallas import tpu as pltpu
from jax.experimental.pallas import tpu_sc as plsc

N_NODES = 10000
M_EDGES = 2500
E_PAIRS = 320000
F_DIM = 128

NC = 2
NS = 16
NW = NC * NS
CH = 80
ROWS_IDX = E_PAIRS // CH
CPW = ROWS_IDX // NW
MPAD = 2560
M_STRIPE = MPAD // NS
NPAD = 10240
N_STRIPE = NPAD // NS


def _matmul(x, w):
    def body(x_ref, w_ref, o_ref):
        o_ref[...] = jnp.dot(x_ref[...], w_ref[...],
                             preferred_element_type=jnp.float32)

    return pl.pallas_call(
        body,
        grid=(10,),
        in_specs=[
            pl.BlockSpec((N_NODES // 10, 128), lambda i: (i, 0)),
            pl.BlockSpec((128, F_DIM), lambda i: (0, 0)),
        ],
        out_specs=pl.BlockSpec((N_NODES // 10, F_DIM), lambda i: (i, 0)),
        out_shape=jax.ShapeDtypeStruct((N_NODES, F_DIM), jnp.float32),
    )(x, w)


_MESH = plsc.VectorSubcoreMesh(core_axis_name="c", subcore_axis_name="s",
                               num_cores=NC, num_subcores=NS)


@functools.partial(
    pl.kernel,
    out_type=jax.ShapeDtypeStruct((NC, MPAD, F_DIM), jnp.float32),
    mesh=_MESH,
    scratch_types=[
        pltpu.VMEM((CPW, CH), jnp.int32),
        pltpu.VMEM((CPW, CH), jnp.int32),
        pltpu.VMEM((CH, F_DIM), jnp.float32),
        pltpu.VMEM_SHARED((MPAD, F_DIM), jnp.float32),
    ],
)
def _stage1(xp_hbm, g1s_hbm, g1d_hbm, zeros_hbm, out_hbm,
            sidx, didx, rows, acc):
    c = lax.axis_index("c")
    s = lax.axis_index("s")
    w = c * NS + s

    pltpu.sync_copy(zeros_hbm.at[pl.ds(s * M_STRIPE, M_STRIPE)],
                    acc.at[pl.ds(s * M_STRIPE, M_STRIPE)])
    pltpu.sync_copy(g1s_hbm.at[w], sidx)
    pltpu.sync_copy(g1d_hbm.at[w], didx)
    plsc.subcore_barrier()

    def chunk(j, carry):
        pltpu.sync_copy(xp_hbm.at[sidx.at[j]], rows)
        pltpu.sync_copy(rows, acc.at[didx.at[j]], add=True)
        return carry

    lax.fori_loop(0, CPW, chunk, 0)
    plsc.subcore_barrier()
    pltpu.sync_copy(acc.at[pl.ds(s * M_STRIPE, M_STRIPE)],
                    out_hbm.at[c, pl.ds(s * M_STRIPE, M_STRIPE)])


def _scale_xe(xep, dege, wmul):
    def body(p_ref, d_ref, w_ref, o_ref):
        o_ref[...] = ((p_ref[0] + p_ref[1]) * d_ref[...] * w_ref[...])

    return pl.pallas_call(
        body,
        in_specs=[
            pl.BlockSpec((NC, MPAD, F_DIM), lambda: (0, 0, 0)),
            pl.BlockSpec((MPAD, 1), lambda: (0, 0)),
            pl.BlockSpec((MPAD, 1), lambda: (0, 0)),
        ],
        out_specs=pl.BlockSpec((MPAD, F_DIM), lambda: (0, 0)),
        out_shape=jax.ShapeDtypeStruct((MPAD, F_DIM), jnp.float32),
    )(xep, dege, wmul)


@functools.partial(
    pl.kernel,
    out_type=jax.ShapeDtypeStruct((NC, NPAD, F_DIM), jnp.float32),
    mesh=_MESH,
    scratch_types=[
        pltpu.VMEM((CPW, CH), jnp.int32),
        pltpu.VMEM((CPW, CH), jnp.int32),
        pltpu.VMEM((CH, F_DIM), jnp.float32),
        pltpu.VMEM_SHARED((NPAD, F_DIM), jnp.float32),
    ],
)
def _stage2(xe_hbm, g2s_hbm, g2d_hbm, zeros_hbm, out_hbm,
            sidx, didx, rows, xv):
    c = lax.axis_index("c")
    s = lax.axis_index("s")
    w = c * NS + s

    pltpu.sync_copy(zeros_hbm.at[pl.ds(s * N_STRIPE, N_STRIPE)],
                    xv.at[pl.ds(s * N_STRIPE, N_STRIPE)])
    pltpu.sync_copy(g2s_hbm.at[w], sidx)
    pltpu.sync_copy(g2d_hbm.at[w], didx)
    plsc.subcore_barrier()

    def chunk(j, carry):
        pltpu.sync_copy(xe_hbm.at[sidx.at[j]], rows)
        pltpu.sync_copy(rows, xv.at[didx.at[j]], add=True)
        return carry

    lax.fori_loop(0, CPW, chunk, 0)
    plsc.subcore_barrier()
    pltpu.sync_copy(xv.at[pl.ds(s * N_STRIPE, N_STRIPE)],
                    out_hbm.at[c, pl.ds(s * N_STRIPE, N_STRIPE)])


def _combine(p0, p1, degv):
    def body(a_ref, b_ref, d_ref, o_ref):
        o_ref[...] = (a_ref[...] + b_ref[...]) * d_ref[...]

    blk = N_NODES // 10
    return pl.pallas_call(
        body,
        grid=(10,),
        in_specs=[
            pl.BlockSpec((blk, F_DIM), lambda i: (i, 0)),
            pl.BlockSpec((blk, F_DIM), lambda i: (i, 0)),
            pl.BlockSpec((blk, 1), lambda i: (i, 0)),
        ],
        out_specs=pl.BlockSpec((blk, F_DIM), lambda i: (i, 0)),
        out_shape=jax.ShapeDtypeStruct((N_NODES, F_DIM), jnp.float32),
    )(p0, p1, degv)


def kernel(X, g1_src, g1_dst, g2_src, g2_dst, W_lin, degE, degV, W):
    xp = _matmul(X, W_lin)

    g1s = g1_src.astype(jnp.int32).reshape(NW, CPW, CH)
    g1d = g1_dst.astype(jnp.int32).reshape(NW, CPW, CH)
    g2s = g2_src.astype(jnp.int32).reshape(NW, CPW, CH)
    g2d = g2_dst.astype(jnp.int32).reshape(NW, CPW, CH)

    zeros_m = jnp.zeros((MPAD, F_DIM), jnp.float32)
    xep = _stage1(xp, g1s, g1d, zeros_m)

    dege_p = jnp.pad(degE, ((0, MPAD - M_EDGES), (0, 0)))
    wmul_p = jnp.pad(W, ((0, MPAD - M_EDGES), (0, 0)))
    xe = _scale_xe(xep, dege_p, wmul_p)
    zeros_n = jnp.zeros((NPAD, F_DIM), jnp.float32)
    xvp = _stage2(xe, g2s, g2d, zeros_n)

    return _combine(xvp[0, :N_NODES], xvp[1, :N_NODES], degV)


# speedup vs baseline: 8.8297x; 1.3236x over previous
import functools

import jax
import jax.numpy as jnp
from jax import lax
from jax.experimental import pallas as pl
from jax.experimental.pallas import tpu as pltpu
from jax.experimental.pallas import tpu_sc as plsc

N_NODES = 10000
M_EDGES = 2500
E_PAIRS = 320000
F_DIM = 128
FH = F_DIM // 2

NC = 2
NS = 16
NW = NC * NS
CH = 125
CPW = E_PAIRS // (NS * CH)
MPAD = 2560
M_STRIPE = MPAD // NS
NPAD = 10240
N_STRIPE = NPAD // NS


def _matmul(x, w):
    def body(x_ref, w_ref, o_ref):
        o_ref[0] = jnp.dot(x_ref[...], w_ref[0],
                           preferred_element_type=jnp.float32)

    blk = N_NODES // 10
    return pl.pallas_call(
        body,
        grid=(2, 10),
        in_specs=[
            pl.BlockSpec((blk, 128), lambda j, i: (i, 0)),
            pl.BlockSpec((1, 128, FH), lambda j, i: (j, 0, 0)),
        ],
        out_specs=pl.BlockSpec((1, blk, FH), lambda j, i: (j, i, 0)),
        out_shape=jax.ShapeDtypeStruct((NC, N_NODES, FH), jnp.float32),
    )(x, w)


def _gather_scatter_loop(table, sidx, didx, rows0, rows1, sem0, sem1, acc):
    bufs = ((rows0, sem0), (rows1, sem1))
    pltpu.async_copy(table.at[sidx.at[0]], rows0, sem0)
    pltpu.async_copy(table.at[sidx.at[1]], rows1, sem1)

    def body(i, carry):
        for b, (rows, sem) in enumerate(bufs):
            j = 2 * i + b
            pltpu.make_async_copy(table.at[sidx.at[j]], rows, sem).wait()
            pltpu.sync_copy(rows, acc.at[didx.at[j]], add=True)
            nxt = j + 2

            @pl.when(nxt < CPW)
            def _():
                pltpu.async_copy(table.at[sidx.at[nxt]], rows, sem)

        return carry

    lax.fori_loop(0, CPW // 2, body, 0)


_MESH = plsc.VectorSubcoreMesh(core_axis_name="c", subcore_axis_name="s",
                               num_cores=NC, num_subcores=NS)


@functools.partial(
    pl.kernel,
    out_type=jax.ShapeDtypeStruct((NC, MPAD, FH), jnp.float32),
    mesh=_MESH,
    compiler_params=pltpu.CompilerParams(use_tc_tiling_on_sc=False),
    scratch_types=[
        pltpu.VMEM((CPW, CH), jnp.int32),
        pltpu.VMEM((CPW, CH), jnp.int32),
        pltpu.VMEM((CH, FH), jnp.float32),
        pltpu.VMEM((CH, FH), jnp.float32),
        pltpu.SemaphoreType.DMA,
        pltpu.SemaphoreType.DMA,
        pltpu.VMEM_SHARED((MPAD, FH), jnp.float32),
    ],
)
def _stage1(xp_hbm, g1s_hbm, g1d_hbm, zeros_hbm, out_hbm,
            sidx, didx, rows0, rows1, sem0, sem1, acc):
    c = lax.axis_index("c")
    s = lax.axis_index("s")

    pltpu.sync_copy(zeros_hbm.at[pl.ds(s * M_STRIPE, M_STRIPE)],
                    acc.at[pl.ds(s * M_STRIPE, M_STRIPE)])
    pltpu.sync_copy(g1s_hbm.at[s], sidx)
    pltpu.sync_copy(g1d_hbm.at[s], didx)
    plsc.subcore_barrier()

    @pl.when(c == 0)
    def _():
        _gather_scatter_loop(xp_hbm.at[0], sidx, didx,
                             rows0, rows1, sem0, sem1, acc)

    @pl.when(c == 1)
    def _():
        _gather_scatter_loop(xp_hbm.at[1], sidx, didx,
                             rows0, rows1, sem0, sem1, acc)

    plsc.subcore_barrier()
    pltpu.sync_copy(acc.at[pl.ds(s * M_STRIPE, M_STRIPE)],
                    out_hbm.at[c, pl.ds(s * M_STRIPE, M_STRIPE)])


def _scale_xe(xep, dege, wmul):
    def body(p_ref, d_ref, w_ref, o_ref):
        o_ref[...] = p_ref[...] * d_ref[...] * w_ref[...]

    return pl.pallas_call(
        body,
        grid=(2,),
        in_specs=[
            pl.BlockSpec((1, MPAD, FH), lambda j: (j, 0, 0)),
            pl.BlockSpec((MPAD, 1), lambda j: (0, 0)),
            pl.BlockSpec((MPAD, 1), lambda j: (0, 0)),
        ],
        out_specs=pl.BlockSpec((1, MPAD, FH), lambda j: (j, 0, 0)),
        out_shape=jax.ShapeDtypeStruct((NC, MPAD, FH), jnp.float32),
    )(xep, dege, wmul)


@functools.partial(
    pl.kernel,
    out_type=jax.ShapeDtypeStruct((NC, NPAD, FH), jnp.float32),
    mesh=_MESH,
    compiler_params=pltpu.CompilerParams(use_tc_tiling_on_sc=False),
    scratch_types=[
        pltpu.VMEM((CPW, CH), jnp.int32),
        pltpu.VMEM((CPW, CH), jnp.int32),
        pltpu.VMEM((CH, FH), jnp.float32),
        pltpu.VMEM((CH, FH), jnp.float32),
        pltpu.SemaphoreType.DMA,
        pltpu.SemaphoreType.DMA,
        pltpu.VMEM_SHARED((MPAD, FH), jnp.float32),
        pltpu.VMEM_SHARED((NPAD, FH), jnp.float32),
    ],
)
def _stage2(xe_hbm, g2s_hbm, g2d_hbm, zeros_hbm, out_hbm,
            sidx, didx, rows0, rows1, sem0, sem1, xe, xv):
    c = lax.axis_index("c")
    s = lax.axis_index("s")

    mbase = s * M_STRIPE
    pltpu.sync_copy(xe_hbm.at[c, pl.ds(mbase, M_STRIPE)],
                    xe.at[pl.ds(mbase, M_STRIPE)])
    pltpu.sync_copy(zeros_hbm.at[pl.ds(s * N_STRIPE, N_STRIPE)],
                    xv.at[pl.ds(s * N_STRIPE, N_STRIPE)])
    pltpu.sync_copy(g2s_hbm.at[s], sidx)
    pltpu.sync_copy(g2d_hbm.at[s], didx)
    plsc.subcore_barrier()

    _gather_scatter_loop(xe, sidx, didx, rows0, rows1, sem0, sem1, xv)

    plsc.subcore_barrier()
    pltpu.sync_copy(xv.at[pl.ds(s * N_STRIPE, N_STRIPE)],
                    out_hbm.at[c, pl.ds(s * N_STRIPE, N_STRIPE)])


def _combine(xvp, degv):
    def body(p_ref, d_ref, o_ref):
        o_ref[:, :FH] = p_ref[0] * d_ref[...]
        o_ref[:, FH:] = p_ref[1] * d_ref[...]

    blk = N_NODES // 10
    return pl.pallas_call(
        body,
        grid=(10,),
        in_specs=[
            pl.BlockSpec((NC, blk, FH), lambda i: (0, i, 0)),
            pl.BlockSpec((blk, 1), lambda i: (i, 0)),
        ],
        out_specs=pl.BlockSpec((blk, F_DIM), lambda i: (i, 0)),
        out_shape=jax.ShapeDtypeStruct((N_NODES, F_DIM), jnp.float32),
    )(xvp, degv)


def kernel(X, g1_src, g1_dst, g2_src, g2_dst, W_lin, degE, degV, W):
    w_split = W_lin.reshape(128, NC, FH).transpose(1, 0, 2)
    xp = _matmul(X, w_split)

    g1s = g1_src.astype(jnp.int32).reshape(NS, CPW, CH)
    g1d = g1_dst.astype(jnp.int32).reshape(NS, CPW, CH)
    g2s = g2_src.astype(jnp.int32).reshape(NS, CPW, CH)
    g2d = g2_dst.astype(jnp.int32).reshape(NS, CPW, CH)

    zeros_m = jnp.zeros((MPAD, FH), jnp.float32)
    xep = _stage1(xp, g1s, g1d, zeros_m)

    dege_p = jnp.pad(degE, ((0, MPAD - M_EDGES), (0, 0)))
    wmul_p = jnp.pad(W, ((0, MPAD - M_EDGES), (0, 0)))
    xe = _scale_xe(xep, dege_p, wmul_p)
    zeros_n = jnp.zeros((NPAD, FH), jnp.float32)
    xvp = _stage2(xe, g2s, g2d, zeros_n)

    return _combine(xvp[:, :N_NODES], degV)


# speedup vs baseline: 8.8849x; 1.0063x over previous
import functools

import jax
import jax.numpy as jnp
from jax import lax
from jax.experimental import pallas as pl
from jax.experimental.pallas import tpu as pltpu
from jax.experimental.pallas import tpu_sc as plsc

N_NODES = 10000
M_EDGES = 2500
E_PAIRS = 320000
F_DIM = 128
FH = F_DIM // 2

NC = 2
NS = 16
NW = NC * NS
CH = 125
CPW = E_PAIRS // (NS * CH)
MPAD = 2560
M_STRIPE = MPAD // NS
NPAD = 10240
N_STRIPE = NPAD // NS


def _matmul(x, w):
    def body(x_ref, w_ref, o_ref):
        o_ref[0] = jnp.dot(x_ref[...], w_ref[0],
                           preferred_element_type=jnp.float32)

    blk = N_NODES // 10
    return pl.pallas_call(
        body,
        grid=(2, 10),
        in_specs=[
            pl.BlockSpec((blk, 128), lambda j, i: (i, 0)),
            pl.BlockSpec((1, 128, FH), lambda j, i: (j, 0, 0)),
        ],
        out_specs=pl.BlockSpec((1, blk, FH), lambda j, i: (j, i, 0)),
        out_shape=jax.ShapeDtypeStruct((NC, N_NODES, FH), jnp.float32),
    )(x, w)


def _gather_scatter_loop(table, sidx, didx, rows0, rows1, sem0, sem1, acc):
    bufs = ((rows0, sem0), (rows1, sem1))
    pltpu.async_copy(table.at[sidx.at[0]], rows0, sem0)
    pltpu.async_copy(table.at[sidx.at[1]], rows1, sem1)

    def body(i, carry):
        for b, (rows, sem) in enumerate(bufs):
            j = 2 * i + b
            pltpu.make_async_copy(table.at[sidx.at[j]], rows, sem).wait()
            pltpu.sync_copy(rows, acc.at[didx.at[j]], add=True)
            nxt = j + 2

            @pl.when(nxt < CPW)
            def _():
                pltpu.async_copy(table.at[sidx.at[nxt]], rows, sem)

        return carry

    lax.fori_loop(0, CPW // 2, body, 0)


_MESH = plsc.VectorSubcoreMesh(core_axis_name="c", subcore_axis_name="s",
                               num_cores=NC, num_subcores=NS)


@functools.partial(
    pl.kernel,
    out_type=jax.ShapeDtypeStruct((NC, NPAD, FH), jnp.float32),
    mesh=_MESH,
    compiler_params=pltpu.CompilerParams(use_tc_tiling_on_sc=False),
    scratch_types=[
        pltpu.VMEM((CPW, CH), jnp.int32),
        pltpu.VMEM((CPW, CH), jnp.int32),
        pltpu.VMEM((CH, FH), jnp.float32),
        pltpu.VMEM((CH, FH), jnp.float32),
        pltpu.SemaphoreType.DMA,
        pltpu.SemaphoreType.DMA,
        pltpu.VMEM((M_STRIPE, FH), jnp.float32),
        pltpu.VMEM((M_STRIPE,), jnp.float32),
        pltpu.VMEM((M_STRIPE,), jnp.float32),
        pltpu.VMEM_SHARED((MPAD, FH), jnp.float32),
        pltpu.VMEM_SHARED((NPAD, FH), jnp.float32),
    ],
)
def _fused(xp_hbm, g1s_hbm, g1d_hbm, g2s_hbm, g2d_hbm, dege_hbm, wmul_hbm,
           zeros_m_hbm, zeros_n_hbm, out_hbm,
           sidx, didx, rows0, rows1, sem0, sem1, xbuf, dbuf, wbuf, xe, xv):
    c = lax.axis_index("c")
    s = lax.axis_index("s")
    mbase = s * M_STRIPE

    pltpu.sync_copy(zeros_m_hbm.at[pl.ds(mbase, M_STRIPE)],
                    xe.at[pl.ds(mbase, M_STRIPE)])
    pltpu.sync_copy(zeros_n_hbm.at[pl.ds(s * N_STRIPE, N_STRIPE)],
                    xv.at[pl.ds(s * N_STRIPE, N_STRIPE)])
    pltpu.sync_copy(g1s_hbm.at[s], sidx)
    pltpu.sync_copy(g1d_hbm.at[s], didx)
    pltpu.sync_copy(dege_hbm.at[pl.ds(mbase, M_STRIPE)], dbuf)
    pltpu.sync_copy(wmul_hbm.at[pl.ds(mbase, M_STRIPE)], wbuf)
    plsc.subcore_barrier()

    @pl.when(c == 0)
    def _():
        _gather_scatter_loop(xp_hbm.at[0], sidx, didx,
                             rows0, rows1, sem0, sem1, xe)

    @pl.when(c == 1)
    def _():
        _gather_scatter_loop(xp_hbm.at[1], sidx, didx,
                             rows0, rows1, sem0, sem1, xe)

    pltpu.sync_copy(g2s_hbm.at[s], sidx)
    pltpu.sync_copy(g2d_hbm.at[s], didx)
    plsc.subcore_barrier()

    pltpu.sync_copy(xe.at[pl.ds(mbase, M_STRIPE)], xbuf)

    def scale_group(g, carry):
        sv = dbuf[pl.ds(g * 16, 16)] * wbuf[pl.ds(g * 16, 16)]
        for k in range(16):
            srow = sv.at[jnp.full((16,), k, jnp.int32)].get(
                mode="promise_in_bounds")
            r = g * 16 + k
            for q in range(FH // 16):
                sl = pl.ds(q * 16, 16)
                xbuf[r, sl] = xbuf[r, sl] * srow
        return carry

    lax.fori_loop(0, M_STRIPE // 16, scale_group, 0)
    pltpu.sync_copy(xbuf, xe.at[pl.ds(mbase, M_STRIPE)])
    plsc.subcore_barrier()

    _gather_scatter_loop(xe, sidx, didx, rows0, rows1, sem0, sem1, xv)

    plsc.subcore_barrier()
    pltpu.sync_copy(xv.at[pl.ds(s * N_STRIPE, N_STRIPE)],
                    out_hbm.at[c, pl.ds(s * N_STRIPE, N_STRIPE)])


def _combine(xvp, degv):
    def body(p_ref, d_ref, o_ref):
        o_ref[:, :FH] = p_ref[0] * d_ref[...]
        o_ref[:, FH:] = p_ref[1] * d_ref[...]

    blk = N_NODES // 10
    return pl.pallas_call(
        body,
        grid=(10,),
        in_specs=[
            pl.BlockSpec((NC, blk, FH), lambda i: (0, i, 0)),
            pl.BlockSpec((blk, 1), lambda i: (i, 0)),
        ],
        out_specs=pl.BlockSpec((blk, F_DIM), lambda i: (i, 0)),
        out_shape=jax.ShapeDtypeStruct((N_NODES, F_DIM), jnp.float32),
    )(xvp, degv)


def kernel(X, g1_src, g1_dst, g2_src, g2_dst, W_lin, degE, degV, W):
    w_split = W_lin.reshape(128, NC, FH).transpose(1, 0, 2)
    xp = _matmul(X, w_split)

    g1s = g1_src.astype(jnp.int32).reshape(NS, CPW, CH)
    g1d = g1_dst.astype(jnp.int32).reshape(NS, CPW, CH)
    g2s = g2_src.astype(jnp.int32).reshape(NS, CPW, CH)
    g2d = g2_dst.astype(jnp.int32).reshape(NS, CPW, CH)

    dege_p = jnp.pad(degE.reshape(M_EDGES), (0, MPAD - M_EDGES))
    wmul_p = jnp.pad(W.reshape(M_EDGES), (0, MPAD - M_EDGES))
    zeros_m = jnp.zeros((MPAD, FH), jnp.float32)
    zeros_n = jnp.zeros((NPAD, FH), jnp.float32)

    xvp = _fused(xp, g1s, g1d, g2s, g2d, dege_p, wmul_p, zeros_m, zeros_n)

    return _combine(xvp[:, :N_NODES], degV)


# speedup vs baseline: 10.2175x; 1.1500x over previous
import functools

import jax
import jax.numpy as jnp
from jax import lax
from jax.experimental import pallas as pl
from jax.experimental.pallas import tpu as pltpu
from jax.experimental.pallas import tpu_sc as plsc

N_NODES = 10000
M_EDGES = 2500
E_PAIRS = 320000
F_DIM = 128
FH = F_DIM // 2

NC = 2
NS = 16
NW = NC * NS
CH = 125
NB = 4
KSLACK = 2
NPASS = 2
CPP = E_PAIRS // (NS * NPASS * CH)
MPAD = 2560
M_STRIPE = MPAD // NS
NPAD = 10240
N_STRIPE = NPAD // NS


def _matmul(x, w):
    def body(x_ref, w_ref, o_ref):
        o_ref[0] = jnp.dot(x_ref[...], w_ref[0],
                           preferred_element_type=jnp.float32)

    blk = N_NODES // 10
    return pl.pallas_call(
        body,
        grid=(2, 10),
        in_specs=[
            pl.BlockSpec((blk, 128), lambda j, i: (i, 0)),
            pl.BlockSpec((1, 128, FH), lambda j, i: (j, 0, 0)),
        ],
        out_specs=pl.BlockSpec((1, blk, FH), lambda j, i: (j, i, 0)),
        out_shape=jax.ShapeDtypeStruct((NC, N_NODES, FH), jnp.float32),
    )(x, w)


def _gs_pass(table, sidx, didx, rows, semg, sems, acc):
    for b in range(NB):
        pltpu.async_copy(table.at[sidx.at[b]], rows[b], semg[b])

    def body(i, carry):
        for b in range(NB):
            j = NB * i + b
            gs = j + KSLACK
            gb = (b + KSLACK) % NB

            @pl.when(jnp.logical_and(gs >= NB, gs < CPP))
            def _():
                pltpu.make_async_copy(rows[gb], acc.at[didx.at[0]],
                                      sems[gb]).wait()
                pltpu.async_copy(table.at[sidx.at[gs]], rows[gb], semg[gb])

            pltpu.make_async_copy(table.at[sidx.at[j]], rows[b],
                                  semg[b]).wait()
            pltpu.async_copy(rows[b], acc.at[didx.at[j]], sems[b], add=True)
        return carry

    lax.fori_loop(0, CPP // NB, body, 0)
    for b in range(NB):
        pltpu.make_async_copy(rows[b], acc.at[didx.at[0]], sems[b]).wait()


_MESH = plsc.VectorSubcoreMesh(core_axis_name="c", subcore_axis_name="s",
                               num_cores=NC, num_subcores=NS)


@functools.partial(
    pl.kernel,
    out_type=jax.ShapeDtypeStruct((NC, NPAD, FH), jnp.float32),
    mesh=_MESH,
    compiler_params=pltpu.CompilerParams(use_tc_tiling_on_sc=False),
    scratch_types=[
        pltpu.VMEM((CPP, CH), jnp.int32),
        pltpu.VMEM((CPP, CH), jnp.int32),
        pltpu.VMEM((CH, FH), jnp.float32),
        pltpu.VMEM((CH, FH), jnp.float32),
        pltpu.VMEM((CH, FH), jnp.float32),
        pltpu.VMEM((CH, FH), jnp.float32),
        pltpu.SemaphoreType.DMA,
        pltpu.SemaphoreType.DMA,
        pltpu.SemaphoreType.DMA,
        pltpu.SemaphoreType.DMA,
        pltpu.SemaphoreType.DMA,
        pltpu.SemaphoreType.DMA,
        pltpu.SemaphoreType.DMA,
        pltpu.SemaphoreType.DMA,
        pltpu.VMEM((M_STRIPE, FH), jnp.float32),
        pltpu.VMEM((M_STRIPE,), jnp.float32),
        pltpu.VMEM((M_STRIPE,), jnp.float32),
        pltpu.VMEM_SHARED((MPAD, FH), jnp.float32),
        pltpu.VMEM_SHARED((NPAD, FH), jnp.float32),
    ],
)
def _fused(xp_hbm, g1s_hbm, g1d_hbm, g2s_hbm, g2d_hbm, dege_hbm, wmul_hbm,
           zeros_m_hbm, zeros_n_hbm, out_hbm,
           sidx, didx, r0, r1, r2, r3, g0, g1, g2, g3, s0, s1, s2, s3,
           xbuf, dbuf, wbuf, xe, xv):
    c = lax.axis_index("c")
    s = lax.axis_index("s")
    mbase = s * M_STRIPE
    rows = (r0, r1, r2, r3)
    semg = (g0, g1, g2, g3)
    sems = (s0, s1, s2, s3)

    pltpu.sync_copy(zeros_m_hbm.at[pl.ds(mbase, M_STRIPE)],
                    xe.at[pl.ds(mbase, M_STRIPE)])
    pltpu.sync_copy(zeros_n_hbm.at[pl.ds(s * N_STRIPE, N_STRIPE)],
                    xv.at[pl.ds(s * N_STRIPE, N_STRIPE)])
    pltpu.sync_copy(dege_hbm.at[pl.ds(mbase, M_STRIPE)], dbuf)
    pltpu.sync_copy(wmul_hbm.at[pl.ds(mbase, M_STRIPE)], wbuf)
    plsc.subcore_barrier()

    for p in range(NPASS):
        pltpu.sync_copy(g1s_hbm.at[s, p], sidx)
        pltpu.sync_copy(g1d_hbm.at[s, p], didx)

        @pl.when(c == 0)
        def _():
            _gs_pass(xp_hbm.at[0], sidx, didx, rows, semg, sems, xe)

        @pl.when(c == 1)
        def _():
            _gs_pass(xp_hbm.at[1], sidx, didx, rows, semg, sems, xe)

    plsc.subcore_barrier()

    pltpu.sync_copy(xe.at[pl.ds(mbase, M_STRIPE)], xbuf)

    def scale_group(g, carry):
        sv = dbuf[pl.ds(g * 16, 16)] * wbuf[pl.ds(g * 16, 16)]
        for k in range(16):
            srow = sv.at[jnp.full((16,), k, jnp.int32)].get(
                mode="promise_in_bounds")
            r = g * 16 + k
            for q in range(FH // 16):
                sl = pl.ds(q * 16, 16)
                xbuf[r, sl] = xbuf[r, sl] * srow
        return carry

    lax.fori_loop(0, M_STRIPE // 16, scale_group, 0)
    pltpu.sync_copy(xbuf, xe.at[pl.ds(mbase, M_STRIPE)])
    plsc.subcore_barrier()

    for p in range(NPASS):
        pltpu.sync_copy(g2s_hbm.at[s, p], sidx)
        pltpu.sync_copy(g2d_hbm.at[s, p], didx)
        _gs_pass(xe, sidx, didx, rows, semg, sems, xv)

    plsc.subcore_barrier()
    pltpu.sync_copy(xv.at[pl.ds(s * N_STRIPE, N_STRIPE)],
                    out_hbm.at[c, pl.ds(s * N_STRIPE, N_STRIPE)])


def _combine(xvp, degv):
    def body(p_ref, d_ref, o_ref):
        o_ref[:, :FH] = p_ref[0] * d_ref[...]
        o_ref[:, FH:] = p_ref[1] * d_ref[...]

    blk = N_NODES // 10
    return pl.pallas_call(
        body,
        grid=(10,),
        in_specs=[
            pl.BlockSpec((NC, blk, FH), lambda i: (0, i, 0)),
            pl.BlockSpec((blk, 1), lambda i: (i, 0)),
        ],
        out_specs=pl.BlockSpec((blk, F_DIM), lambda i: (i, 0)),
        out_shape=jax.ShapeDtypeStruct((N_NODES, F_DIM), jnp.float32),
    )(xvp, degv)


def kernel(X, g1_src, g1_dst, g2_src, g2_dst, W_lin, degE, degV, W):
    w_split = W_lin.reshape(128, NC, FH).transpose(1, 0, 2)
    xp = _matmul(X, w_split)

    g1s = g1_src.astype(jnp.int32).reshape(NS, NPASS, CPP, CH)
    g1d = g1_dst.astype(jnp.int32).reshape(NS, NPASS, CPP, CH)
    g2s = g2_src.astype(jnp.int32).reshape(NS, NPASS, CPP, CH)
    g2d = g2_dst.astype(jnp.int32).reshape(NS, NPASS, CPP, CH)

    dege_p = jnp.pad(degE.reshape(M_EDGES), (0, MPAD - M_EDGES))
    wmul_p = jnp.pad(W.reshape(M_EDGES), (0, MPAD - M_EDGES))
    zeros_m = jnp.zeros((MPAD, FH), jnp.float32)
    zeros_n = jnp.zeros((NPAD, FH), jnp.float32)

    xvp = _fused(xp, g1s, g1d, g2s, g2d, dege_p, wmul_p, zeros_m, zeros_n)

    return _combine(xvp[:, :N_NODES], degV)
